# baseline (speedup 1.0000x reference)
import jax
import jax.numpy as jnp
from jax import lax
from jax.experimental import pallas as pl
from jax.experimental.pallas import tpu as pltpu
from jax.experimental.pallas import tpu_sc as plsc

_CH = 64
_LANES = 16


def _sc_update(idx_hbm, agent_hbm, grid_ref, idx_v, agent_v, vals_v, sem):
    wid = lax.axis_index("s") * 2 + lax.axis_index("c")

    @pl.when(wid == 0)
    def _():
        pltpu.sync_copy(idx_hbm, idx_v)
        pltpu.sync_copy(agent_hbm, agent_v)
        pltpu.async_copy(grid_ref.at[idx_v], vals_v, sem).wait()
        for j in range(_CH // _LANES):
            sl = pl.ds(j * _LANES, _LANES)
            vals_v[sl] = jnp.maximum(vals_v[sl], agent_v[sl])
        pltpu.async_copy(vals_v, grid_ref.at[idx_v], sem).wait()


def kernel(input_grid, input_state_of_agent, coordinates_at_last_frame, scene_id):
    s, ch, h, w = input_grid.shape
    c0 = coordinates_at_last_frame[0].astype(jnp.int32)
    c1 = coordinates_at_last_frame[1].astype(jnp.int32)
    sid = jnp.asarray(scene_id, jnp.int32)
    base = sid * (ch * h * w) + c0 * w + c1
    idx = base + jnp.arange(ch, dtype=jnp.int32) * (h * w)
    agent = input_state_of_agent.reshape(ch).astype(jnp.float32)

    grid_ref = jax.new_ref(input_grid.reshape(-1))

    sc_kernel = pl.kernel(
        _sc_update,
        out_type=(),
        mesh=plsc.VectorSubcoreMesh(
            core_axis_name="c", subcore_axis_name="s",
            num_cores=2, num_subcores=16,
        ),
        scratch_types=[
            pltpu.VMEM((ch,), jnp.int32),
            pltpu.VMEM((ch,), jnp.float32),
            pltpu.VMEM((ch,), jnp.float32),
            pltpu.SemaphoreType.DMA,
        ],
    )
    sc_kernel(idx, agent, grid_ref)
    return jax.freeze(grid_ref).reshape(s, ch, h, w)
